# baseline (speedup 1.0000x reference)
import functools
import math

import jax
import jax.numpy as jnp
from jax import lax
from jax.experimental import pallas as pl
from jax.experimental.pallas import tpu as pltpu
from jax.experimental.pallas import tpu_sc as plsc

EMB = 32
SCALE = math.sqrt(EMB)

NC, NS = 2, 16
NW = NC * NS
TB = 128
TC = 4


def _make_kernel(B0, T):
    blocks_per_w = B0 // (NW * TB)
    n_pairs = (T // TC) // 2
    tpad = -(-T // 16) * 16
    mesh = plsc.VectorSubcoreMesh(core_axis_name="c", subcore_axis_name="s")

    @functools.partial(
        pl.kernel,
        mesh=mesh,
        out_type=jax.ShapeDtypeStruct((T, EMB // 8, B0 // TB, 8, TB),
                                      jnp.float32),
        scratch_types=[
            pltpu.VMEM((TB, tpad), jnp.int32),
            pltpu.VMEM((T * TB,), jnp.int32),
            pltpu.VMEM((TC * TB, EMB), jnp.float32),
            pltpu.VMEM((TC * TB, EMB), jnp.float32),
            pltpu.VMEM((TC, EMB // 8, 8, TB), jnp.float32),
            pltpu.VMEM((TC, EMB // 8, 8, TB), jnp.float32),
            pltpu.SemaphoreType.DMA,
            pltpu.SemaphoreType.DMA,
            pltpu.SemaphoreType.DMA,
            pltpu.SemaphoreType.DMA,
        ],
        compiler_params=pltpu.CompilerParams(use_tc_tiling_on_sc=False,
                                             needs_layout_passes=False),
    )
    def body_kernel(tok_hbm, tab_hbm, out_hbm,
                    tokb, tokt, rows0, rows1, rt0, rt1,
                    sem_g0, sem_g1, sem_o0, sem_o1):
        wid = lax.axis_index("s") * NC + lax.axis_index("c")
        lanes = lax.iota(jnp.int32, 16)
        rowsel = [lanes + g * 16 for g in range(TB // 16)]

        def transpose_scale(rows, rt):
            def ebody(e, carry):
                r, es = e // 8, e % 8
                evec = jnp.full((16,), e, jnp.int32)
                for tt in range(TC):
                    for g in range(TB // 16):
                        v = plsc.load_gather(
                            rows, [rowsel[g] + tt * TB, evec])
                        rt[tt, r, es, pl.ds(g * 16, 16)] = v * SCALE
                return carry
            lax.fori_loop(0, EMB, ebody, 0)

        def start_gather(tc, rows, sem):
            return pltpu.async_copy(
                tab_hbm.at[tokt.at[pl.ds(tc * (TC * TB), TC * TB)]],
                rows, sem)

        def write_out(t0, bblk, rt, sem):
            for tt in range(TC):
                pltpu.async_copy(rt.at[tt], out_hbm.at[t0 + tt, :, bblk],
                                 sem)

        def wait_out(rt, sem):
            for tt in range(TC):
                pltpu.make_async_copy(rt.at[tt], out_hbm.at[0, :, 0],
                                      sem).wait()

        def block_body(kb, carry):
            b0 = wid * (blocks_per_w * TB) + kb * TB
            bblk = wid * blocks_per_w + kb
            pltpu.sync_copy(tok_hbm.at[pl.ds(b0, TB)],
                            tokb.at[:, pl.ds(0, T)])

            def tbody(i, carry):
                tvec = lanes + i * 16
                msk = tvec < T
                for b in range(TB):
                    v = tokb[b, pl.ds(i * 16, 16)]
                    plsc.store_scatter(tokt, [tvec * TB + b], v, mask=msk)
                return carry

            lax.fori_loop(0, tpad // 16, tbody, 0)

            g0 = start_gather(0, rows0, sem_g0)

            def pair_body(g, carry):
                tc0 = 2 * g
                start_gather(tc0 + 1, rows1, sem_g1)
                pltpu.make_async_copy(
                    tab_hbm.at[tokt.at[pl.ds(0, TC * TB)]],
                    rows0, sem_g0).wait()

                @pl.when(g > 0)
                def _():
                    wait_out(rt0, sem_o0)
                transpose_scale(rows0, rt0)
                write_out(tc0 * TC, bblk, rt0, sem_o0)
                @pl.when(g < n_pairs - 1)
                def _():
                    start_gather(tc0 + 2, rows0, sem_g0)
                pltpu.make_async_copy(
                    tab_hbm.at[tokt.at[pl.ds(0, TC * TB)]],
                    rows1, sem_g1).wait()

                @pl.when(g > 0)
                def _():
                    wait_out(rt1, sem_o1)
                transpose_scale(rows1, rt1)
                write_out((tc0 + 1) * TC, bblk, rt1, sem_o1)
                return carry

            lax.fori_loop(0, n_pairs, pair_body, 0)
            wait_out(rt0, sem_o0)
            wait_out(rt1, sem_o1)
            return carry

        lax.fori_loop(0, blocks_per_w, block_body, 0)

    return body_kernel


def kernel(tokens, table):
    B0, T = tokens.shape
    o5 = _make_kernel(B0, T)(tokens.astype(jnp.int32), table)
    y = jnp.transpose(o5, (2, 4, 0, 1, 3))
    return y.reshape(B0, T, EMB)


# speedup vs baseline: 2.0190x; 2.0190x over previous
import functools
import math

import jax
import jax.numpy as jnp
from jax import lax
from jax.experimental import pallas as pl
from jax.experimental.pallas import tpu as pltpu
from jax.experimental.pallas import tpu_sc as plsc

EMB = 32
SCALE = math.sqrt(EMB)

NC, NS = 2, 16
NW = NC * NS
TB = 128
TC = 4


def _make_kernel(B0, T):
    blocks_per_w = B0 // (NW * TB)
    n_pairs = (T // TC) // 2
    tpad = -(-T // 16) * 16
    mesh = plsc.VectorSubcoreMesh(core_axis_name="c", subcore_axis_name="s")

    @functools.partial(
        pl.kernel,
        mesh=mesh,
        out_type=jax.ShapeDtypeStruct((T, EMB // 8, B0 // TB, 8, TB),
                                      jnp.float32),
        scratch_types=[
            pltpu.VMEM((TB, tpad), jnp.int32),
            pltpu.VMEM((T * TB,), jnp.int32),
            pltpu.VMEM((TC * TB, EMB), jnp.float32),
            pltpu.VMEM((TC * TB, EMB), jnp.float32),
            pltpu.VMEM((TC, EMB // 8, 8, TB + 1), jnp.float32),
            pltpu.VMEM((TC, EMB // 8, 8, TB + 1), jnp.float32),
            pltpu.SemaphoreType.DMA,
            pltpu.SemaphoreType.DMA,
            pltpu.SemaphoreType.DMA,
            pltpu.SemaphoreType.DMA,
        ],
        compiler_params=pltpu.CompilerParams(use_tc_tiling_on_sc=False,
                                             needs_layout_passes=False),
    )
    def body_kernel(tok_hbm, tab_hbm, out_hbm,
                    tokb, tokt, rows0, rows1, rt0, rt1,
                    sem_g0, sem_g1, sem_o0, sem_o1):
        wid = lax.axis_index("s") * NC + lax.axis_index("c")
        lanes = lax.iota(jnp.int32, 16)
        rowsel = [lanes + g * 16 for g in range(TB // 16)]

        rvec = [(lanes + h) // 8 for h in (0, 16)]
        esvec = [(lanes + h) % 8 for h in (0, 16)]
        ttvec = [jnp.full((16,), tt, jnp.int32) for tt in range(TC)]

        def transpose_scale(rows, rt):
            def bbody(i, carry):
                for q in range(4):
                    b = i * 4 + q
                    bvec = jnp.full((16,), 0, jnp.int32) + b
                    for tt in range(TC):
                        for hi, h in enumerate((0, 16)):
                            v = rows[tt * TB + b, pl.ds(h, 16)]
                            plsc.store_scatter(
                                rt, [ttvec[tt], rvec[hi], esvec[hi], bvec],
                                v * SCALE)
                return carry
            lax.fori_loop(0, TB // 4, bbody, 0)

        def start_gather(tc, rows, sem):
            return pltpu.async_copy(
                tab_hbm.at[tokt.at[pl.ds(tc * (TC * TB), TC * TB)]],
                rows, sem)

        def write_out(t0, bblk, rt, sem):
            for tt in range(TC):
                pltpu.async_copy(rt.at[tt, :, :, pl.ds(0, TB)],
                                 out_hbm.at[t0 + tt, :, bblk], sem)

        def wait_out(rt, sem):
            for tt in range(TC):
                pltpu.make_async_copy(rt.at[tt, :, :, pl.ds(0, TB)],
                                      out_hbm.at[0, :, 0], sem).wait()

        def block_body(kb, carry):
            b0 = wid * (blocks_per_w * TB) + kb * TB
            bblk = wid * blocks_per_w + kb
            pltpu.sync_copy(tok_hbm.at[pl.ds(b0, TB)],
                            tokb.at[:, pl.ds(0, T)])

            def tbody(i, carry):
                tvec = lanes + i * 16
                msk = tvec < T
                for b in range(TB):
                    v = tokb[b, pl.ds(i * 16, 16)]
                    plsc.store_scatter(tokt, [tvec * TB + b], v, mask=msk)
                return carry

            lax.fori_loop(0, tpad // 16, tbody, 0)

            g0 = start_gather(0, rows0, sem_g0)

            def pair_body(g, carry):
                tc0 = 2 * g
                start_gather(tc0 + 1, rows1, sem_g1)
                pltpu.make_async_copy(
                    tab_hbm.at[tokt.at[pl.ds(0, TC * TB)]],
                    rows0, sem_g0).wait()

                @pl.when(g > 0)
                def _():
                    wait_out(rt0, sem_o0)
                transpose_scale(rows0, rt0)
                write_out(tc0 * TC, bblk, rt0, sem_o0)
                @pl.when(g < n_pairs - 1)
                def _():
                    start_gather(tc0 + 2, rows0, sem_g0)
                pltpu.make_async_copy(
                    tab_hbm.at[tokt.at[pl.ds(0, TC * TB)]],
                    rows1, sem_g1).wait()

                @pl.when(g > 0)
                def _():
                    wait_out(rt1, sem_o1)
                transpose_scale(rows1, rt1)
                write_out((tc0 + 1) * TC, bblk, rt1, sem_o1)
                return carry

            lax.fori_loop(0, n_pairs, pair_body, 0)
            wait_out(rt0, sem_o0)
            wait_out(rt1, sem_o1)
            return carry

        lax.fori_loop(0, blocks_per_w, block_body, 0)

    return body_kernel


def kernel(tokens, table):
    B0, T = tokens.shape
    o5 = _make_kernel(B0, T)(tokens.astype(jnp.int32), table)
    y = jnp.transpose(o5, (2, 4, 0, 1, 3))
    return y.reshape(B0, T, EMB)
